# baseline (speedup 1.0000x reference)
import functools
import jax
import jax.numpy as jnp
from jax import lax
from jax.experimental import pallas as pl
from jax.experimental.pallas import tpu as pltpu, tpu_sc as plsc

BATCH = 4096
FIELDS = 26
NVAL = 1000
NROWS = BATCH * FIELDS
NC = 2
NS = 16
NW = NC * NS
ROWS_W = NROWS // NW
CHUNK = 32
CWORDS = CHUNK * NVAL
NCHUNK = ROWS_W // CHUNK
NGRP = (CHUNK + 15) // 16


def _sc_body(idx_hbm, out_hbm, idx_v, bufs, sems):
    wid = lax.axis_index("s") * NC + lax.axis_index("c")
    row0 = wid * ROWS_W

    pltpu.sync_copy(idx_hbm.at[pl.ds(row0, ROWS_W)], idx_v)

    zeros16 = jnp.zeros((16,), jnp.float32)
    ones16 = jnp.full((16,), 1.0, jnp.float32)
    lanes = lax.iota(jnp.int32, 16)

    def zero_grp(j, carry):
        bufs[pl.ds(j * 16, 16)] = zeros16
        return carry

    lax.fori_loop(0, 2 * CWORDS // 16, zero_grp, 0)

    def paint(ci, b, val16):
        for g in range(NGRP):
            lane = lanes + g * 16
            col = plsc.load_gather(idx_v, [ci * CHUNK + lane])
            plsc.store_scatter(bufs, [b * CWORDS + lane * NVAL + col], val16)

    def pair(i, carry):
        for b in range(2):
            ci = i * 2 + b

            @pl.when(i > 0)
            def _wait_and_reset():
                pltpu.make_async_copy(
                    bufs.at[pl.ds(b * CWORDS, CWORDS)],
                    out_hbm.at[pl.ds((row0 + (ci - 2) * CHUNK) * NVAL, CWORDS)],
                    sems.at[b],
                ).wait()
                paint(ci - 2, b, zeros16)

            paint(ci, b, ones16)
            pltpu.make_async_copy(
                bufs.at[pl.ds(b * CWORDS, CWORDS)],
                out_hbm.at[pl.ds((row0 + ci * CHUNK) * NVAL, CWORDS)],
                sems.at[b],
            ).start()
        return carry

    lax.fori_loop(0, NCHUNK // 2, pair, 0)

    for b in range(2):
        ci = NCHUNK - 2 + b
        pltpu.make_async_copy(
            bufs.at[pl.ds(b * CWORDS, CWORDS)],
            out_hbm.at[pl.ds((row0 + ci * CHUNK) * NVAL, CWORDS)],
            sems.at[b],
        ).wait()


_sc_one_hot = functools.partial(
    pl.kernel,
    out_type=jax.ShapeDtypeStruct((NROWS * NVAL,), jnp.float32),
    mesh=plsc.VectorSubcoreMesh(core_axis_name="c", subcore_axis_name="s"),
    compiler_params=pltpu.CompilerParams(needs_layout_passes=False),
    scratch_types=[
        pltpu.VMEM((ROWS_W,), jnp.int32),
        pltpu.VMEM((2 * CWORDS,), jnp.float32),
        pltpu.SemaphoreType.DMA((2,)),
    ],
)(_sc_body)


def kernel(input, eye):
    idx = input.astype(jnp.int32).reshape(NROWS)
    out = _sc_one_hot(idx)
    return out.reshape(BATCH, FIELDS, NVAL)


# speedup vs baseline: 1.9644x; 1.9644x over previous
import functools
import jax
import jax.numpy as jnp
from jax import lax
from jax.experimental import pallas as pl
from jax.experimental.pallas import tpu as pltpu, tpu_sc as plsc

BATCH = 4096
FIELDS = 26
NVAL = 1000
NC = 2
NS = 16
NW = NC * NS
BROWS_W = BATCH // NW
NGRP = (FIELDS + 15) // 16


def _sc_body(idx_hbm, out_hbm, idx_v, buf0, buf1, sems):
    wid = lax.axis_index("s") * NC + lax.axis_index("c")
    b0 = wid * BROWS_W

    pltpu.sync_copy(idx_hbm.at[pl.ds(b0 * FIELDS, BROWS_W * FIELDS)], idx_v)

    bufs = (buf0, buf1)
    zeros16 = jnp.zeros((16,), jnp.float32)
    ones16 = jnp.full((16,), 1.0, jnp.float32)
    lanes = lax.iota(jnp.int32, 16)

    def zero_grp(j, carry):
        f = j // (NVAL // 16 + 1)
        g = j - f * (NVAL // 16 + 1)
        off = jnp.minimum(g * 16, NVAL - 16)
        for b in range(2):
            bufs[b][f, pl.ds(off, 16)] = zeros16
        return carry

    lax.fori_loop(0, FIELDS * (NVAL // 16 + 1), zero_grp, 0)

    def paint(ri, b, val16):
        for g in range(NGRP):
            lane = lanes + g * 16
            mask = lane < FIELDS
            f = jnp.minimum(lane, FIELDS - 1)
            col = plsc.load_gather(idx_v, [ri * FIELDS + f], mask=mask)
            plsc.store_scatter(bufs[b], [f, col], val16, mask=mask)

    def pair(i, carry):
        for b in range(2):
            ri = i * 2 + b

            @pl.when(i > 0)
            def _wait_and_reset():
                pltpu.make_async_copy(
                    bufs[b],
                    out_hbm.at[b0 + ri - 2],
                    sems.at[b],
                ).wait()
                paint(ri - 2, b, zeros16)

            paint(ri, b, ones16)
            pltpu.make_async_copy(
                bufs[b],
                out_hbm.at[b0 + ri],
                sems.at[b],
            ).start()
        return carry

    lax.fori_loop(0, BROWS_W // 2, pair, 0)

    for b in range(2):
        ri = BROWS_W - 2 + b
        pltpu.make_async_copy(
            bufs[b],
            out_hbm.at[b0 + ri],
            sems.at[b],
        ).wait()


_sc_one_hot = functools.partial(
    pl.kernel,
    out_type=jax.ShapeDtypeStruct((BATCH, FIELDS, NVAL), jnp.float32),
    mesh=plsc.VectorSubcoreMesh(core_axis_name="c", subcore_axis_name="s"),
    compiler_params=pltpu.CompilerParams(needs_layout_passes=False),
    scratch_types=[
        pltpu.VMEM((BROWS_W * FIELDS,), jnp.int32),
        pltpu.VMEM((FIELDS, NVAL), jnp.float32),
        pltpu.VMEM((FIELDS, NVAL), jnp.float32),
        pltpu.SemaphoreType.DMA((2,)),
    ],
)(_sc_body)


def kernel(input, eye):
    idx = input.astype(jnp.int32).reshape(BATCH * FIELDS)
    return _sc_one_hot(idx)
